# baseline (speedup 1.0000x reference)
import functools

import jax
import jax.numpy as jnp
from jax import lax
from jax.experimental import pallas as pl
from jax.experimental.pallas import tpu as pltpu

_VOCAB = 100000
_CTX = 200
_ND = 32
_HID = 128

_VB = 20000
_NB = _VOCAB // _VB
_KS = 4
_VS = _VB // _KS


_GS = 8


def _gather_body(x_sref, *refs):
    i = pl.program_id(0)
    out_ref = refs[_GS]
    for k in range(_GS):
        r = x_sref[_GS * i + k] % 8
        out_ref[pl.ds(k, 1), :] = refs[k][pl.ds(r, 1), :]


def _tc_gather(table, idx):
    grid_spec = pltpu.PrefetchScalarGridSpec(
        num_scalar_prefetch=1,
        grid=(_CTX // _GS,),
        in_specs=[
            pl.BlockSpec((8, _ND),
                         functools.partial(
                             lambda k, i, xr: (xr[_GS * i + k] // 8, 0), k))
            for k in range(_GS)
        ],
        out_specs=pl.BlockSpec((_GS, _ND), lambda i, xr: (i, 0)),
    )
    return pl.pallas_call(
        _gather_body,
        grid_spec=grid_spec,
        out_shape=jax.ShapeDtypeStruct((_CTX, _ND), jnp.float32),
    )(idx, *([table] * _GS))


def _mlp_body(emb_ref, w1_ref, b1_ref, w2a_ref, w2b_ref, w2c_ref, w2d_ref,
              b2_ref, out_ref, h_ref, m_ref):
    i = pl.program_id(0)

    @pl.when(i == 0)
    def _():
        pre = lax.dot_general(emb_ref[...], w1_ref[...],
                              (((1,), (1,)), ((), ())),
                              preferred_element_type=jnp.float32)
        h_ref[...] = jnp.maximum(pre + b1_ref[...], 0.0)
        m_ref[0] = jnp.float32(-jnp.inf)

    @pl.when(i < _NB)
    def _():
        parts = [
            lax.dot_general(h_ref[...], w_ref[...],
                            (((1,), (1,)), ((), ())),
                            preferred_element_type=jnp.float32)
            for w_ref in (w2a_ref, w2b_ref, w2c_ref, w2d_ref)
        ]
        logits = jnp.concatenate(parts, axis=1) + b2_ref[pl.ds(i, 1), :]
        out_ref[pl.ds(i, 1), :] = logits
        m_ref[0] = jnp.maximum(m_ref[0], jnp.max(logits))

    @pl.when(i == _NB)
    def _():
        m = m_ref[0]
        allv = out_ref[...]
        lse = m + jnp.log(jnp.sum(jnp.exp(allv - m)))
        out_ref[...] = allv - lse


def _tc_mlp(emb, W1, b1, W2, b2):
    return pl.pallas_call(
        _mlp_body,
        grid=(_NB + 1,),
        in_specs=[
            pl.BlockSpec((1, _CTX * _ND), lambda i: (0, 0)),
            pl.BlockSpec((_HID, _CTX * _ND), lambda i: (0, 0)),
            pl.BlockSpec((1, _HID), lambda i: (0, 0)),
        ] + [
            pl.BlockSpec((_VS, _HID),
                         functools.partial(
                             lambda k, i: (_KS * jnp.minimum(i, _NB - 1) + k, 0), k))
            for k in range(_KS)
        ] + [
            pl.BlockSpec((_NB, _VB), lambda i: (0, 0)),
        ],
        out_specs=pl.BlockSpec((_NB, _VB), lambda i: (0, 0)),
        out_shape=jax.ShapeDtypeStruct((_NB, _VB), jnp.float32),
        scratch_shapes=[
            pltpu.VMEM((1, _HID), jnp.float32),
            pltpu.SMEM((1,), jnp.float32),
        ],
        compiler_params=pltpu.CompilerParams(
            dimension_semantics=("arbitrary",)),
    )(emb, W1, b1, W2, W2, W2, W2, b2)


def kernel(x, emb_table, W1, b1, W2, b2):
    rows = _tc_gather(emb_table, x.astype(jnp.int32))
    emb = rows.reshape(1, _CTX * _ND)
    out = _tc_mlp(emb, W1, b1.reshape(1, _HID), W2,
                  b2.reshape(_NB, _VB))
    return out.reshape(1, _VOCAB)


# speedup vs baseline: 1.1559x; 1.1559x over previous
import functools

import jax
import jax.numpy as jnp
from jax import lax
from jax.experimental import pallas as pl
from jax.experimental.pallas import tpu as pltpu

_VOCAB = 100000
_CTX = 200
_ND = 32
_HID = 128

_VB = 20000
_NB = _VOCAB // _VB
_KS = 4
_VS = _VB // _KS


def _gather_body(idx_ref, table_ref, out_ref, sem):
    copies = [
        pltpu.make_async_copy(
            table_ref.at[pl.ds(idx_ref[j], 1), :],
            out_ref.at[pl.ds(j, 1), :], sem)
        for j in range(_CTX)
    ]
    for c in copies:
        c.start()
    for c in copies:
        c.wait()


def _tc_gather(table, idx):
    return pl.pallas_call(
        _gather_body,
        in_specs=[
            pl.BlockSpec(memory_space=pltpu.SMEM),
            pl.BlockSpec(memory_space=pltpu.MemorySpace.HBM),
        ],
        out_specs=pl.BlockSpec(memory_space=pltpu.MemorySpace.HBM),
        out_shape=jax.ShapeDtypeStruct((_CTX, _ND), jnp.float32),
        scratch_shapes=[pltpu.SemaphoreType.DMA],
    )(idx, table)


def _mlp_body(emb_ref, w1_ref, b1_ref, w2a_ref, w2b_ref, w2c_ref, w2d_ref,
              b2_ref, out_ref, h_ref, m_ref):
    i = pl.program_id(0)

    @pl.when(i == 0)
    def _():
        pre = lax.dot_general(emb_ref[...], w1_ref[...],
                              (((1,), (1,)), ((), ())),
                              preferred_element_type=jnp.float32)
        h_ref[...] = jnp.maximum(pre + b1_ref[...], 0.0).astype(jnp.bfloat16)
        m_ref[0] = jnp.float32(-jnp.inf)

    @pl.when(i < _NB)
    def _():
        hb = h_ref[...]
        parts = [
            lax.dot_general(hb, w_ref[...].astype(jnp.bfloat16),
                            (((1,), (1,)), ((), ())),
                            preferred_element_type=jnp.float32)
            for w_ref in (w2a_ref, w2b_ref, w2c_ref, w2d_ref)
        ]
        logits = jnp.concatenate(parts, axis=1) + b2_ref[pl.ds(i, 1), :]
        out_ref[pl.ds(i, 1), :] = logits
        m_ref[0] = jnp.maximum(m_ref[0], jnp.max(logits))

    @pl.when(i == _NB)
    def _():
        m = m_ref[0]
        allv = out_ref[...]
        lse = m + jnp.log(jnp.sum(jnp.exp(allv - m)))
        out_ref[...] = allv - lse


def _tc_mlp(emb, W1, b1, W2, b2):
    return pl.pallas_call(
        _mlp_body,
        grid=(_NB + 1,),
        in_specs=[
            pl.BlockSpec((1, _CTX * _ND), lambda i: (0, 0)),
            pl.BlockSpec((_HID, _CTX * _ND), lambda i: (0, 0)),
            pl.BlockSpec((1, _HID), lambda i: (0, 0)),
        ] + [
            pl.BlockSpec((_VS, _HID),
                         functools.partial(
                             lambda k, i: (_KS * jnp.minimum(i, _NB - 1) + k, 0), k))
            for k in range(_KS)
        ] + [
            pl.BlockSpec((_NB, _VB), lambda i: (0, 0)),
        ],
        out_specs=pl.BlockSpec((_NB, _VB), lambda i: (0, 0)),
        out_shape=jax.ShapeDtypeStruct((_NB, _VB), jnp.float32),
        scratch_shapes=[
            pltpu.VMEM((1, _HID), jnp.bfloat16),
            pltpu.SMEM((1,), jnp.float32),
        ],
        compiler_params=pltpu.CompilerParams(
            dimension_semantics=("arbitrary",)),
    )(emb, W1, b1, W2, W2, W2, W2, b2)


def kernel(x, emb_table, W1, b1, W2, b2):
    rows = _tc_gather(emb_table, x.astype(jnp.int32))
    emb = rows.reshape(1, _CTX * _ND)
    out = _tc_mlp(emb, W1, b1.reshape(1, _HID), W2,
                  b2.reshape(_NB, _VB))
    return out.reshape(1, _VOCAB)
